# Initial kernel scaffold; baseline (speedup 1.0000x reference)
import functools

import jax
import jax.numpy as jnp
from jax import lax
from jax.experimental import pallas as pl
from jax.experimental.pallas import tpu as pltpu
from jax.experimental.pallas import tpu_sc as plsc

N = 100000
E = 3200000
DIM = 128
NC = 2
NS = 16
NW = NC * NS
NPAD = 100352
SLICE = NPAD // NS
EPW = E // NW
CH = 2000
NCHUNK = EPW // CH
RBLK = 1024
NBLK = NPAD // RBLK


def _k1_body(src_hbm, z1_hbm, ones_hbm, ho_hbm, ones_v, idx_v, zbuf, hist_sh):
    c = lax.axis_index("c")
    s = lax.axis_index("s")
    w = c * NS + s
    pltpu.sync_copy(z1_hbm.at[pl.ds(0, SLICE)], zbuf)
    pltpu.sync_copy(zbuf, hist_sh.at[pl.ds(s * SLICE, SLICE)])
    pltpu.sync_copy(ones_hbm, ones_v)
    plsc.subcore_barrier()
    base = w * EPW

    def chunk(i, carry):
        pltpu.sync_copy(src_hbm.at[pl.ds(base + i * CH, CH)], idx_v)
        pltpu.sync_copy(ones_v, hist_sh.at[idx_v], add=True)
        return carry

    lax.fori_loop(0, NCHUNK, chunk, 0)
    plsc.subcore_barrier()
    pltpu.sync_copy(hist_sh.at[pl.ds(s * SLICE, SLICE)], zbuf)
    pltpu.sync_copy(zbuf, ho_hbm.at[pl.ds(c * NPAD + s * SLICE, SLICE)])


def _hist(src, z1, ones_c):
    mesh = plsc.VectorSubcoreMesh(core_axis_name="c", subcore_axis_name="s")
    f = pl.kernel(
        _k1_body,
        out_type=jax.ShapeDtypeStruct((NC * NPAD,), jnp.float32),
        mesh=mesh,
        scratch_types=[
            pltpu.VMEM((CH,), jnp.float32),
            pltpu.VMEM((CH,), jnp.int32),
            pltpu.VMEM((SLICE,), jnp.float32),
            pltpu.VMEM_SHARED((NPAD,), jnp.float32),
        ],
    )
    return f(src, z1, ones_c)


def _k3_body(src_hbm, dst_hbm, v_hbm, z4_hbm, agg_hbm,
             sidx, didx, rows, zbuf4, agg_sh, sem):
    c = lax.axis_index("c")
    s = lax.axis_index("s")
    w = c * NS + s
    pltpu.sync_copy(z4_hbm.at[pl.ds(0, SLICE)], zbuf4)
    pltpu.sync_copy(zbuf4, agg_sh.at[pl.ds(s * SLICE, SLICE)])
    plsc.subcore_barrier()
    base = w * EPW

    def chunk(i, carry):
        pltpu.sync_copy(src_hbm.at[pl.ds(base + i * CH, CH)], sidx)
        pltpu.sync_copy(dst_hbm.at[pl.ds(base + i * CH, CH)], didx)
        pltpu.async_copy(v_hbm.at[sidx], rows, sem).wait()
        pltpu.sync_copy(rows, agg_sh.at[didx], add=True)
        return carry

    lax.fori_loop(0, NCHUNK, chunk, 0)
    plsc.subcore_barrier()
    pltpu.sync_copy(agg_sh.at[pl.ds(s * SLICE, SLICE)], zbuf4)
    pltpu.sync_copy(zbuf4, agg_hbm.at[pl.ds(c * NPAD + s * SLICE, SLICE)])


def _scatter(src, dst, v, z4):
    mesh = plsc.VectorSubcoreMesh(core_axis_name="c", subcore_axis_name="s")
    f = pl.kernel(
        _k3_body,
        out_type=jax.ShapeDtypeStruct((NC * NPAD, 4), jnp.float32),
        mesh=mesh,
        scratch_types=[
            pltpu.VMEM((CH,), jnp.int32),
            pltpu.VMEM((CH,), jnp.int32),
            pltpu.VMEM((CH, 4), jnp.float32),
            pltpu.VMEM((SLICE, 4), jnp.float32),
            pltpu.VMEM_SHARED((NPAD, 4), jnp.float32),
            pltpu.SemaphoreType.DMA,
        ],
    )
    return f(src, dst, v, z4)


def _k2_body(ho0_ref, ho1_ref, nw_ref, sg_ref, emb_ref, v_ref):
    od = jnp.maximum(ho0_ref[...] + ho1_ref[...], 1.0)
    sc = lax.rsqrt(od)
    sg = sg_ref[...]
    e00 = emb_ref[0, 0]
    e01 = emb_ref[0, 1]
    e10 = emb_ref[1, 0]
    e11 = emb_ref[1, 1]
    v_ref[:, 0:1] = nw_ref[...] * sc
    v_ref[:, 1:2] = (e00 + (e10 - e00) * sg) * sc
    v_ref[:, 2:3] = (e01 + (e11 - e01) * sg) * sc
    v_ref[:, 3:4] = jnp.ones_like(sc)


def _table(ho0, ho1, nwp, sgp, emb):
    col = pl.BlockSpec((RBLK, 1), lambda i: (i, 0))
    return pl.pallas_call(
        _k2_body,
        grid=(NBLK,),
        in_specs=[col, col, col, col, pl.BlockSpec((2, 2), lambda i: (0, 0))],
        out_specs=pl.BlockSpec((RBLK, 4), lambda i: (i, 0)),
        out_shape=jax.ShapeDtypeStruct((NPAD, 4), jnp.float32),
    )(ho0, ho1, nwp, sgp, emb)


def _k4_body(a0_ref, a1_ref, w_ref, b_ref, o_ref):
    a = a0_ref[...] + a1_ref[...]
    r = lax.rsqrt(jnp.maximum(a[:, 3:4], 1.0))
    w = w_ref[...]
    o_ref[...] = ((a[:, 0:1] * r) * w[0:1, :]
                  + (a[:, 1:2] * r) * w[1:2, :]
                  + (a[:, 2:3] * r) * w[2:3, :]
                  + b_ref[...])


def _project(a0, a1, W0, b0r):
    blk = pl.BlockSpec((RBLK, 4), lambda i: (i, 0))
    return pl.pallas_call(
        _k4_body,
        grid=(NBLK,),
        in_specs=[blk, blk,
                  pl.BlockSpec((3, DIM), lambda i: (0, 0)),
                  pl.BlockSpec((1, DIM), lambda i: (0, 0))],
        out_specs=pl.BlockSpec((RBLK, DIM), lambda i: (i, 0)),
        out_shape=jax.ShapeDtypeStruct((NPAD, DIM), jnp.float32),
    )(a0, a1, W0, b0r)


def kernel(significance, node_weight, edge_index, emb_table, W0, b0):
    src = edge_index[0].astype(jnp.int32)
    dst = edge_index[1].astype(jnp.int32)
    nwp = jnp.pad(node_weight.astype(jnp.float32), (0, NPAD - N)).reshape(NPAD, 1)
    sgp = jnp.pad(significance.astype(jnp.float32), (0, NPAD - N)).reshape(NPAD, 1)
    z1 = jnp.zeros((NPAD,), jnp.float32)
    z4 = jnp.zeros((NPAD, 4), jnp.float32)
    ones_c = jnp.ones((CH,), jnp.float32)

    ho = _hist(src, z1, ones_c)
    v = _table(ho[:NPAD].reshape(NPAD, 1), ho[NPAD:].reshape(NPAD, 1),
               nwp, sgp, emb_table.astype(jnp.float32))
    agg = _scatter(src, dst, v, z4)
    out = _project(agg[:NPAD], agg[NPAD:],
                   W0.astype(jnp.float32),
                   b0.astype(jnp.float32).reshape(1, DIM))
    return out[:N]


# speedup vs baseline: 17.6051x; 17.6051x over previous
import jax
import jax.numpy as jnp
from jax import lax
from jax.experimental import pallas as pl
from jax.experimental.pallas import tpu as pltpu
from jax.experimental.pallas import tpu_sc as plsc

N = 100000
E = 3200000
DIM = 128
NC = 2
NS = 16
NW = NC * NS
NPAD = 100352
SLICE = NPAD // NS
EPW = E // NW
CH = 2000
NCHUNK = EPW // CH
RBLK = 1024
NBLK = NPAD // RBLK

_SC_PARAMS = pltpu.CompilerParams(use_tc_tiling_on_sc=False)


def _k1_body(src_hbm, z1_hbm, ones_hbm, ho_hbm, ones_v, idx_v, zbuf, hist_sh):
    c = lax.axis_index("c")
    s = lax.axis_index("s")
    w = c * NS + s
    pltpu.sync_copy(z1_hbm.at[pl.ds(0, SLICE)], zbuf)
    pltpu.sync_copy(zbuf, hist_sh.at[pl.ds(s * SLICE, SLICE)])
    pltpu.sync_copy(ones_hbm, ones_v)
    plsc.subcore_barrier()
    base = w * EPW

    def chunk(i, carry):
        pltpu.sync_copy(src_hbm.at[pl.ds(base + i * CH, CH)], idx_v)
        pltpu.sync_copy(ones_v, hist_sh.at[idx_v], add=True)
        return carry

    lax.fori_loop(0, NCHUNK, chunk, 0)
    plsc.subcore_barrier()
    pltpu.sync_copy(hist_sh.at[pl.ds(s * SLICE, SLICE)], zbuf)
    pltpu.sync_copy(zbuf, ho_hbm.at[pl.ds(c * NPAD + s * SLICE, SLICE)])


def _hist(src, z1, ones_c):
    mesh = plsc.VectorSubcoreMesh(core_axis_name="c", subcore_axis_name="s")
    f = pl.kernel(
        _k1_body,
        out_type=jax.ShapeDtypeStruct((NC * NPAD,), jnp.float32),
        mesh=mesh,
        scratch_types=[
            pltpu.VMEM((CH,), jnp.float32),
            pltpu.VMEM((CH,), jnp.int32),
            pltpu.VMEM((SLICE,), jnp.float32),
            pltpu.VMEM_SHARED((NPAD,), jnp.float32),
        ],
        compiler_params=_SC_PARAMS,
    )
    return f(src, z1, ones_c)


def _k3_body(src_hbm, dst_hbm, v0_hbm, v1_hbm, v2_hbm, z1_hbm, ones_hbm,
             agg_hbm,
             sidx, didx, g0, g1, g2, ones_v, zbuf,
             a0_sh, a1_sh, a2_sh, a3_sh, sem):
    c = lax.axis_index("c")
    s = lax.axis_index("s")
    w = c * NS + s
    pltpu.sync_copy(z1_hbm.at[pl.ds(0, SLICE)], zbuf)
    for a_sh in (a0_sh, a1_sh, a2_sh, a3_sh):
        pltpu.sync_copy(zbuf, a_sh.at[pl.ds(s * SLICE, SLICE)])
    pltpu.sync_copy(ones_hbm, ones_v)
    plsc.subcore_barrier()
    base = w * EPW

    def chunk(i, carry):
        off = base + i * CH
        pltpu.sync_copy(src_hbm.at[pl.ds(off, CH)], sidx)
        pltpu.sync_copy(dst_hbm.at[pl.ds(off, CH)], didx)
        d0 = pltpu.async_copy(v0_hbm.at[sidx], g0, sem)
        d1 = pltpu.async_copy(v1_hbm.at[sidx], g1, sem)
        d2 = pltpu.async_copy(v2_hbm.at[sidx], g2, sem)
        d0.wait()
        d1.wait()
        d2.wait()
        pltpu.sync_copy(g0, a0_sh.at[didx], add=True)
        pltpu.sync_copy(g1, a1_sh.at[didx], add=True)
        pltpu.sync_copy(g2, a2_sh.at[didx], add=True)
        pltpu.sync_copy(ones_v, a3_sh.at[didx], add=True)
        return carry

    lax.fori_loop(0, NCHUNK, chunk, 0)
    plsc.subcore_barrier()
    for ch, a_sh in enumerate((a0_sh, a1_sh, a2_sh, a3_sh)):
        pltpu.sync_copy(a_sh.at[pl.ds(s * SLICE, SLICE)], zbuf)
        pltpu.sync_copy(zbuf,
                        agg_hbm.at[pl.ds((c * 4 + ch) * NPAD + s * SLICE,
                                         SLICE)])


def _scatter(src, dst, v0, v1, v2, z1, ones_c):
    mesh = plsc.VectorSubcoreMesh(core_axis_name="c", subcore_axis_name="s")
    f = pl.kernel(
        _k3_body,
        out_type=jax.ShapeDtypeStruct((NC * 4 * NPAD,), jnp.float32),
        mesh=mesh,
        scratch_types=[
            pltpu.VMEM((CH,), jnp.int32),
            pltpu.VMEM((CH,), jnp.int32),
            pltpu.VMEM((CH,), jnp.float32),
            pltpu.VMEM((CH,), jnp.float32),
            pltpu.VMEM((CH,), jnp.float32),
            pltpu.VMEM((CH,), jnp.float32),
            pltpu.VMEM((SLICE,), jnp.float32),
            pltpu.VMEM_SHARED((NPAD,), jnp.float32),
            pltpu.VMEM_SHARED((NPAD,), jnp.float32),
            pltpu.VMEM_SHARED((NPAD,), jnp.float32),
            pltpu.VMEM_SHARED((NPAD,), jnp.float32),
            pltpu.SemaphoreType.DMA,
        ],
        compiler_params=_SC_PARAMS,
    )
    return f(src, dst, v0, v1, v2, z1, ones_c)


def _k2_body(ho0_ref, ho1_ref, nw_ref, sg_ref, emb_ref, v0_ref, v1_ref, v2_ref):
    od = jnp.maximum(ho0_ref[...] + ho1_ref[...], 1.0)
    sc = lax.rsqrt(od)
    sg = sg_ref[...]
    e00 = emb_ref[0, 0]
    e01 = emb_ref[0, 1]
    e10 = emb_ref[1, 0]
    e11 = emb_ref[1, 1]
    v0_ref[...] = nw_ref[...] * sc
    v1_ref[...] = (e00 + (e10 - e00) * sg) * sc
    v2_ref[...] = (e01 + (e11 - e01) * sg) * sc


def _table(ho0, ho1, nwp, sgp, emb):
    col = pl.BlockSpec((RBLK, 1), lambda i: (i, 0))
    shp = jax.ShapeDtypeStruct((NPAD, 1), jnp.float32)
    return pl.pallas_call(
        _k2_body,
        grid=(NBLK,),
        in_specs=[col, col, col, col, pl.BlockSpec((2, 2), lambda i: (0, 0))],
        out_specs=[col, col, col],
        out_shape=[shp, shp, shp],
    )(ho0, ho1, nwp, sgp, emb)


def _k4_body(q0_ref, f10_ref, f20_ref, c0_ref, q1_ref, f11_ref, f21_ref,
             c1_ref, w_ref, b_ref, o_ref):
    q = q0_ref[...] + q1_ref[...]
    f1 = f10_ref[...] + f11_ref[...]
    f2 = f20_ref[...] + f21_ref[...]
    cnt = c0_ref[...] + c1_ref[...]
    r = lax.rsqrt(jnp.maximum(cnt, 1.0))
    w = w_ref[...]
    o_ref[...] = ((q * r) * w[0:1, :]
                  + (f1 * r) * w[1:2, :]
                  + (f2 * r) * w[2:3, :]
                  + b_ref[...])


def _project(parts, W0, b0r):
    col = pl.BlockSpec((RBLK, 1), lambda i: (i, 0))
    return pl.pallas_call(
        _k4_body,
        grid=(NBLK,),
        in_specs=[col] * 8 + [pl.BlockSpec((3, DIM), lambda i: (0, 0)),
                              pl.BlockSpec((1, DIM), lambda i: (0, 0))],
        out_specs=pl.BlockSpec((RBLK, DIM), lambda i: (i, 0)),
        out_shape=jax.ShapeDtypeStruct((NPAD, DIM), jnp.float32),
    )(*parts, W0, b0r)


def kernel(significance, node_weight, edge_index, emb_table, W0, b0):
    src = edge_index[0].astype(jnp.int32)
    dst = edge_index[1].astype(jnp.int32)
    nwp = jnp.pad(node_weight.astype(jnp.float32), (0, NPAD - N)).reshape(NPAD, 1)
    sgp = jnp.pad(significance.astype(jnp.float32), (0, NPAD - N)).reshape(NPAD, 1)
    z1 = jnp.zeros((NPAD,), jnp.float32)
    ones_c = jnp.ones((CH,), jnp.float32)

    ho = _hist(src, z1, ones_c)
    v0, v1, v2 = _table(ho[:NPAD].reshape(NPAD, 1), ho[NPAD:].reshape(NPAD, 1),
                        nwp, sgp, emb_table.astype(jnp.float32))
    agg = _scatter(src, dst, v0.reshape(NPAD), v1.reshape(NPAD),
                   v2.reshape(NPAD), z1, ones_c)
    a = agg.reshape(NC * 4, NPAD, 1)
    out = _project([a[i] for i in range(8)],
                   W0.astype(jnp.float32),
                   b0.astype(jnp.float32).reshape(1, DIM))
    return out[:N]


# speedup vs baseline: 18.8725x; 1.0720x over previous
import jax
import jax.numpy as jnp
from jax import lax
from jax.experimental import pallas as pl
from jax.experimental.pallas import tpu as pltpu
from jax.experimental.pallas import tpu_sc as plsc

N = 100000
E = 3200000
DIM = 128
NC = 2
NS = 16
NW = NC * NS
NPAD = 100352
SLICE = NPAD // NS
EPW = E // NW
CH = 10000
CH1 = 50000
NCHUNK = EPW // CH
RBLK = 1024
NBLK = NPAD // RBLK

_SC_PARAMS = pltpu.CompilerParams(use_tc_tiling_on_sc=False)


def _k1_body(src_hbm, z1_hbm, ones_hbm, ho_hbm, ones_v, idx_v, zbuf, hist_sh):
    c = lax.axis_index("c")
    s = lax.axis_index("s")
    w = c * NS + s
    pltpu.sync_copy(z1_hbm.at[pl.ds(0, SLICE)], zbuf)
    pltpu.sync_copy(zbuf, hist_sh.at[pl.ds(s * SLICE, SLICE)])
    pltpu.sync_copy(ones_hbm, ones_v)
    plsc.subcore_barrier()
    base = w * EPW

    def chunk(i, carry):
        pltpu.sync_copy(src_hbm.at[pl.ds(base + i * CH1, CH1)], idx_v)
        pltpu.sync_copy(ones_v, hist_sh.at[idx_v], add=True)
        return carry

    lax.fori_loop(0, EPW // CH1, chunk, 0)
    plsc.subcore_barrier()
    pltpu.sync_copy(hist_sh.at[pl.ds(s * SLICE, SLICE)], zbuf)
    pltpu.sync_copy(zbuf, ho_hbm.at[pl.ds(c * NPAD + s * SLICE, SLICE)])


def _hist(src, z1, ones_c):
    mesh = plsc.VectorSubcoreMesh(core_axis_name="c", subcore_axis_name="s")
    f = pl.kernel(
        _k1_body,
        out_type=jax.ShapeDtypeStruct((NC * NPAD,), jnp.float32),
        mesh=mesh,
        scratch_types=[
            pltpu.VMEM((CH1,), jnp.float32),
            pltpu.VMEM((CH1,), jnp.int32),
            pltpu.VMEM((SLICE,), jnp.float32),
            pltpu.VMEM_SHARED((NPAD,), jnp.float32),
        ],
        compiler_params=_SC_PARAMS,
    )
    return f(src, z1, ones_c)


def _k3_body(src_hbm, dst_hbm, v0_hbm, v1_hbm, v2_hbm, z1_hbm, ones_hbm,
             agg_hbm,
             sidx, didx, g0, g1, g2, ones_v, zbuf,
             a0_sh, a1_sh, a2_sh, a3_sh, sem):
    c = lax.axis_index("c")
    s = lax.axis_index("s")
    w = c * NS + s
    pltpu.sync_copy(z1_hbm.at[pl.ds(0, SLICE)], zbuf)
    for a_sh in (a0_sh, a1_sh, a2_sh, a3_sh):
        pltpu.sync_copy(zbuf, a_sh.at[pl.ds(s * SLICE, SLICE)])
    pltpu.sync_copy(ones_hbm, ones_v)
    plsc.subcore_barrier()
    base = w * EPW

    def chunk(i, carry):
        off = base + i * CH
        pltpu.sync_copy(src_hbm.at[pl.ds(off, CH)], sidx)
        pltpu.sync_copy(dst_hbm.at[pl.ds(off, CH)], didx)
        d0 = pltpu.async_copy(v0_hbm.at[sidx], g0, sem)
        d1 = pltpu.async_copy(v1_hbm.at[sidx], g1, sem)
        d2 = pltpu.async_copy(v2_hbm.at[sidx], g2, sem)
        d0.wait()
        d1.wait()
        d2.wait()
        pltpu.sync_copy(g0, a0_sh.at[didx], add=True)
        pltpu.sync_copy(g1, a1_sh.at[didx], add=True)
        pltpu.sync_copy(g2, a2_sh.at[didx], add=True)
        pltpu.sync_copy(ones_v, a3_sh.at[didx], add=True)
        return carry

    lax.fori_loop(0, NCHUNK, chunk, 0)
    plsc.subcore_barrier()
    for ch, a_sh in enumerate((a0_sh, a1_sh, a2_sh, a3_sh)):
        pltpu.sync_copy(a_sh.at[pl.ds(s * SLICE, SLICE)], zbuf)
        pltpu.sync_copy(zbuf,
                        agg_hbm.at[pl.ds((c * 4 + ch) * NPAD + s * SLICE,
                                         SLICE)])


def _scatter(src, dst, v0, v1, v2, z1, ones_c):
    mesh = plsc.VectorSubcoreMesh(core_axis_name="c", subcore_axis_name="s")
    f = pl.kernel(
        _k3_body,
        out_type=jax.ShapeDtypeStruct((NC * 4 * NPAD,), jnp.float32),
        mesh=mesh,
        scratch_types=[
            pltpu.VMEM((CH,), jnp.int32),
            pltpu.VMEM((CH,), jnp.int32),
            pltpu.VMEM((CH,), jnp.float32),
            pltpu.VMEM((CH,), jnp.float32),
            pltpu.VMEM((CH,), jnp.float32),
            pltpu.VMEM((CH,), jnp.float32),
            pltpu.VMEM((SLICE,), jnp.float32),
            pltpu.VMEM_SHARED((NPAD,), jnp.float32),
            pltpu.VMEM_SHARED((NPAD,), jnp.float32),
            pltpu.VMEM_SHARED((NPAD,), jnp.float32),
            pltpu.VMEM_SHARED((NPAD,), jnp.float32),
            pltpu.SemaphoreType.DMA,
        ],
        compiler_params=_SC_PARAMS,
    )
    return f(src, dst, v0, v1, v2, z1, ones_c)


def _k2_body(ho0_ref, ho1_ref, nw_ref, sg_ref, emb_ref, v0_ref, v1_ref, v2_ref):
    od = jnp.maximum(ho0_ref[...] + ho1_ref[...], 1.0)
    sc = lax.rsqrt(od)
    sg = sg_ref[...]
    e00 = emb_ref[0, 0]
    e01 = emb_ref[0, 1]
    e10 = emb_ref[1, 0]
    e11 = emb_ref[1, 1]
    v0_ref[...] = nw_ref[...] * sc
    v1_ref[...] = (e00 + (e10 - e00) * sg) * sc
    v2_ref[...] = (e01 + (e11 - e01) * sg) * sc


def _table(ho0, ho1, nwp, sgp, emb):
    col = pl.BlockSpec((RBLK, 1), lambda i: (i, 0))
    shp = jax.ShapeDtypeStruct((NPAD, 1), jnp.float32)
    return pl.pallas_call(
        _k2_body,
        grid=(NBLK,),
        in_specs=[col, col, col, col, pl.BlockSpec((2, 2), lambda i: (0, 0))],
        out_specs=[col, col, col],
        out_shape=[shp, shp, shp],
    )(ho0, ho1, nwp, sgp, emb)


def _k4_body(q0_ref, f10_ref, f20_ref, c0_ref, q1_ref, f11_ref, f21_ref,
             c1_ref, w_ref, b_ref, o_ref):
    q = q0_ref[...] + q1_ref[...]
    f1 = f10_ref[...] + f11_ref[...]
    f2 = f20_ref[...] + f21_ref[...]
    cnt = c0_ref[...] + c1_ref[...]
    r = lax.rsqrt(jnp.maximum(cnt, 1.0))
    w = w_ref[...]
    o_ref[...] = ((q * r) * w[0:1, :]
                  + (f1 * r) * w[1:2, :]
                  + (f2 * r) * w[2:3, :]
                  + b_ref[...])


def _project(parts, W0, b0r):
    col = pl.BlockSpec((RBLK, 1), lambda i: (i, 0))
    return pl.pallas_call(
        _k4_body,
        grid=(NBLK,),
        in_specs=[col] * 8 + [pl.BlockSpec((3, DIM), lambda i: (0, 0)),
                              pl.BlockSpec((1, DIM), lambda i: (0, 0))],
        out_specs=pl.BlockSpec((RBLK, DIM), lambda i: (i, 0)),
        out_shape=jax.ShapeDtypeStruct((NPAD, DIM), jnp.float32),
    )(*parts, W0, b0r)


def kernel(significance, node_weight, edge_index, emb_table, W0, b0):
    src = edge_index[0].astype(jnp.int32)
    dst = edge_index[1].astype(jnp.int32)
    nwp = jnp.pad(node_weight.astype(jnp.float32), (0, NPAD - N)).reshape(NPAD, 1)
    sgp = jnp.pad(significance.astype(jnp.float32), (0, NPAD - N)).reshape(NPAD, 1)
    z1 = jnp.zeros((NPAD,), jnp.float32)
    ones_c = jnp.ones((CH,), jnp.float32)
    ones_c1 = jnp.ones((CH1,), jnp.float32)

    ho = _hist(src, z1, ones_c1)
    v0, v1, v2 = _table(ho[:NPAD].reshape(NPAD, 1), ho[NPAD:].reshape(NPAD, 1),
                        nwp, sgp, emb_table.astype(jnp.float32))
    agg = _scatter(src, dst, v0.reshape(NPAD), v1.reshape(NPAD),
                   v2.reshape(NPAD), z1, ones_c)
    a = agg.reshape(NC * 4, NPAD, 1)
    out = _project([a[i] for i in range(8)],
                   W0.astype(jnp.float32),
                   b0.astype(jnp.float32).reshape(1, DIM))
    return out[:N]


# speedup vs baseline: 32.8554x; 1.7409x over previous
import jax
import jax.numpy as jnp
from jax import lax
from jax.experimental import pallas as pl
from jax.experimental.pallas import tpu as pltpu
from jax.experimental.pallas import tpu_sc as plsc

N = 100000
E = 3200000
DIM = 128
NC = 2
NS = 16
NW = NC * NS
NPAD = 100352
SLICE = NPAD // NS
EPW = E // NW
CH = 10000
CH1 = 50000
NCHUNK = EPW // CH
RBLK = 1024
NBLK = NPAD // RBLK

_SC_PARAMS = pltpu.CompilerParams(use_tc_tiling_on_sc=False)


def _k1_body(src_hbm, z1_hbm, ones_hbm, ho_hbm, ones_v, idx_v, zbuf, hist_sh):
    c = lax.axis_index("c")
    s = lax.axis_index("s")
    w = c * NS + s
    pltpu.sync_copy(z1_hbm.at[pl.ds(0, SLICE)], zbuf)
    pltpu.sync_copy(zbuf, hist_sh.at[pl.ds(s * SLICE, SLICE)])
    pltpu.sync_copy(ones_hbm, ones_v)
    plsc.subcore_barrier()
    base = w * EPW

    def chunk(i, carry):
        pltpu.sync_copy(src_hbm.at[pl.ds(base + i * CH1, CH1)], idx_v)
        pltpu.sync_copy(ones_v, hist_sh.at[idx_v], add=True)
        return carry

    lax.fori_loop(0, EPW // CH1, chunk, 0)
    plsc.subcore_barrier()
    pltpu.sync_copy(hist_sh.at[pl.ds(s * SLICE, SLICE)], zbuf)
    pltpu.sync_copy(zbuf, ho_hbm.at[pl.ds(c * NPAD + s * SLICE, SLICE)])


def _hist(src, z1, ones_c):
    mesh = plsc.VectorSubcoreMesh(core_axis_name="c", subcore_axis_name="s")
    f = pl.kernel(
        _k1_body,
        out_type=jax.ShapeDtypeStruct((NC * NPAD,), jnp.float32),
        mesh=mesh,
        scratch_types=[
            pltpu.VMEM((CH1,), jnp.float32),
            pltpu.VMEM((CH1,), jnp.int32),
            pltpu.VMEM((SLICE,), jnp.float32),
            pltpu.VMEM_SHARED((NPAD,), jnp.float32),
        ],
        compiler_params=_SC_PARAMS,
    )
    return f(src, z1, ones_c)


def _k3_body(src_hbm, dst_hbm, v0_hbm, v1_hbm, v2_hbm, z1_hbm, ones_hbm,
             agg_hbm,
             sidx, didx, g0, g1, g2, ones_v, zbuf,
             a0_sh, a1_sh, a2_sh, a3_sh, sem):
    c = lax.axis_index("c")
    s = lax.axis_index("s")
    w = c * NS + s
    pltpu.sync_copy(z1_hbm.at[pl.ds(0, SLICE)], zbuf)
    for a_sh in (a0_sh, a1_sh, a2_sh, a3_sh):
        pltpu.sync_copy(zbuf, a_sh.at[pl.ds(s * SLICE, SLICE)])
    pltpu.sync_copy(ones_hbm, ones_v)
    plsc.subcore_barrier()
    base = w * EPW

    def chunk(i, carry):
        off = base + i * CH
        pltpu.sync_copy(src_hbm.at[pl.ds(off, CH)], sidx)
        pltpu.sync_copy(dst_hbm.at[pl.ds(off, CH)], didx)
        d0 = pltpu.async_copy(v0_hbm.at[sidx], g0, sem)
        d1 = pltpu.async_copy(v1_hbm.at[sidx], g1, sem)
        d2 = pltpu.async_copy(v2_hbm.at[sidx], g2, sem)
        d0.wait()
        d1.wait()
        d2.wait()
        pltpu.sync_copy(g0, a0_sh.at[didx], add=True)
        pltpu.sync_copy(g1, a1_sh.at[didx], add=True)
        pltpu.sync_copy(g2, a2_sh.at[didx], add=True)
        pltpu.sync_copy(ones_v, a3_sh.at[didx], add=True)
        return carry

    lax.fori_loop(0, NCHUNK, chunk, 0)
    plsc.subcore_barrier()
    for ch, a_sh in enumerate((a0_sh, a1_sh, a2_sh, a3_sh)):
        pltpu.sync_copy(a_sh.at[pl.ds(s * SLICE, SLICE)], zbuf)
        pltpu.sync_copy(zbuf,
                        agg_hbm.at[pl.ds((c * 4 + ch) * NPAD + s * SLICE,
                                         SLICE)])


def _scatter(src, dst, v0, v1, v2, z1, ones_c):
    mesh = plsc.VectorSubcoreMesh(core_axis_name="c", subcore_axis_name="s")
    f = pl.kernel(
        _k3_body,
        out_type=jax.ShapeDtypeStruct((NC * 4 * NPAD,), jnp.float32),
        mesh=mesh,
        scratch_types=[
            pltpu.VMEM((CH,), jnp.int32),
            pltpu.VMEM((CH,), jnp.int32),
            pltpu.VMEM((CH,), jnp.float32),
            pltpu.VMEM((CH,), jnp.float32),
            pltpu.VMEM((CH,), jnp.float32),
            pltpu.VMEM((CH,), jnp.float32),
            pltpu.VMEM((SLICE,), jnp.float32),
            pltpu.VMEM_SHARED((NPAD,), jnp.float32),
            pltpu.VMEM_SHARED((NPAD,), jnp.float32),
            pltpu.VMEM_SHARED((NPAD,), jnp.float32),
            pltpu.VMEM_SHARED((NPAD,), jnp.float32),
            pltpu.SemaphoreType.DMA,
        ],
        compiler_params=_SC_PARAMS,
    )
    return f(src, dst, v0, v1, v2, z1, ones_c)


def _k2_body(ho0_ref, ho1_ref, nw_ref, sg_ref, emb_ref, v0_ref, v1_ref, v2_ref):
    od = jnp.maximum(ho0_ref[...] + ho1_ref[...], 1.0)
    sc = lax.rsqrt(od)
    sg = sg_ref[...]
    e00 = emb_ref[0, 0]
    e01 = emb_ref[0, 1]
    e10 = emb_ref[1, 0]
    e11 = emb_ref[1, 1]
    v0_ref[...] = nw_ref[...] * sc
    v1_ref[...] = (e00 + (e10 - e00) * sg) * sc
    v2_ref[...] = (e01 + (e11 - e01) * sg) * sc


ROWS = NPAD // DIM


def _table(ho0, ho1, nwp, sgp, emb):
    full = pl.BlockSpec((ROWS, DIM), lambda i: (0, 0))
    shp = jax.ShapeDtypeStruct((ROWS, DIM), jnp.float32)
    return pl.pallas_call(
        _k2_body,
        grid=(1,),
        in_specs=[full, full, full, full, pl.BlockSpec((2, 2), lambda i: (0, 0))],
        out_specs=[full, full, full],
        out_shape=[shp, shp, shp],
    )(ho0, ho1, nwp, sgp, emb)


PACK = 8


def _k4_body(q0_ref, f10_ref, f20_ref, c0_ref, q1_ref, f11_ref, f21_ref,
             c1_ref, w_ref, b_ref, o_ref):
    q = q0_ref[...] + q1_ref[...]
    f1 = f10_ref[...] + f11_ref[...]
    f2 = f20_ref[...] + f21_ref[...]
    cnt = c0_ref[...] + c1_ref[...]
    r = lax.rsqrt(jnp.maximum(cnt, 1.0))
    w = w_ref[...]
    t = ((q * r)[:, :, None] * w[0].reshape(1, 1, DIM)
         + (f1 * r)[:, :, None] * w[1].reshape(1, 1, DIM)
         + (f2 * r)[:, :, None] * w[2].reshape(1, 1, DIM)
         + b_ref[...].reshape(1, 1, DIM))
    o_ref[...] = t.reshape(RBLK, DIM)


def _project(parts, W0, b0r):
    pk = pl.BlockSpec((PACK, DIM), lambda i: (i, 0))
    return pl.pallas_call(
        _k4_body,
        grid=(NBLK,),
        in_specs=[pk] * 8 + [pl.BlockSpec((3, DIM), lambda i: (0, 0)),
                             pl.BlockSpec((1, DIM), lambda i: (0, 0))],
        out_specs=pl.BlockSpec((RBLK, DIM), lambda i: (i, 0)),
        out_shape=jax.ShapeDtypeStruct((N, DIM), jnp.float32),
    )(*parts, W0, b0r)


def kernel(significance, node_weight, edge_index, emb_table, W0, b0):
    src = edge_index[0].astype(jnp.int32)
    dst = edge_index[1].astype(jnp.int32)
    nwp = jnp.pad(node_weight.astype(jnp.float32), (0, NPAD - N)).reshape(ROWS, DIM)
    sgp = jnp.pad(significance.astype(jnp.float32), (0, NPAD - N)).reshape(ROWS, DIM)
    z1 = jnp.zeros((NPAD,), jnp.float32)
    ones_c = jnp.ones((CH,), jnp.float32)
    ones_c1 = jnp.ones((CH1,), jnp.float32)

    ho = _hist(src, z1, ones_c1)
    v0, v1, v2 = _table(ho[:NPAD].reshape(ROWS, DIM), ho[NPAD:].reshape(ROWS, DIM),
                        nwp, sgp, emb_table.astype(jnp.float32))
    agg = _scatter(src, dst, v0.reshape(NPAD), v1.reshape(NPAD),
                   v2.reshape(NPAD), z1, ones_c)
    a = agg.reshape(NC * 4, ROWS, DIM)
    out = _project([a[i] for i in range(8)],
                   W0.astype(jnp.float32),
                   b0.astype(jnp.float32).reshape(1, DIM))
    return out


# speedup vs baseline: 37.8937x; 1.1533x over previous
import jax
import jax.numpy as jnp
from jax import lax
from jax.experimental import pallas as pl
from jax.experimental.pallas import tpu as pltpu
from jax.experimental.pallas import tpu_sc as plsc

N = 100000
E = 3200000
DIM = 128
NC = 2
NS = 16
NW = NC * NS
NPAD = 100352
SLICE = NPAD // NS
EPW = E // NW
CH = 5000
CH1 = 50000
NCHUNK = EPW // CH
RBLK = 1024
NBLK = NPAD // RBLK

_SC_PARAMS = pltpu.CompilerParams(use_tc_tiling_on_sc=False)


def _k1_body(src_hbm, z1_hbm, ones_hbm, ho_hbm, ones_v, idx_v, zbuf, hist_sh):
    c = lax.axis_index("c")
    s = lax.axis_index("s")
    w = c * NS + s
    pltpu.sync_copy(z1_hbm.at[pl.ds(0, SLICE)], zbuf)
    pltpu.sync_copy(zbuf, hist_sh.at[pl.ds(s * SLICE, SLICE)])
    pltpu.sync_copy(ones_hbm, ones_v)
    plsc.subcore_barrier()
    base = w * EPW

    def chunk(i, carry):
        pltpu.sync_copy(src_hbm.at[pl.ds(base + i * CH1, CH1)], idx_v)
        pltpu.sync_copy(ones_v, hist_sh.at[idx_v], add=True)
        return carry

    lax.fori_loop(0, EPW // CH1, chunk, 0)
    plsc.subcore_barrier()
    pltpu.sync_copy(hist_sh.at[pl.ds(s * SLICE, SLICE)], zbuf)
    pltpu.sync_copy(zbuf, ho_hbm.at[pl.ds(c * NPAD + s * SLICE, SLICE)])


def _hist(src, z1, ones_c):
    mesh = plsc.VectorSubcoreMesh(core_axis_name="c", subcore_axis_name="s")
    f = pl.kernel(
        _k1_body,
        out_type=jax.ShapeDtypeStruct((NC * NPAD,), jnp.float32),
        mesh=mesh,
        scratch_types=[
            pltpu.VMEM((CH1,), jnp.float32),
            pltpu.VMEM((CH1,), jnp.int32),
            pltpu.VMEM((SLICE,), jnp.float32),
            pltpu.VMEM_SHARED((NPAD,), jnp.float32),
        ],
        compiler_params=_SC_PARAMS,
    )
    return f(src, z1, ones_c)


def _k3_body(src_hbm, dst_hbm, v0_hbm, v1_hbm, v2_hbm, z1_hbm, ones_hbm,
             agg_hbm,
             sidx0, sidx1, didx0, didx1,
             g00, g10, g20, g01, g11, g21, ones_v, zbuf,
             a0_sh, a1_sh, a2_sh, a3_sh,
             semg0, semg1, sems0, sems1):
    c = lax.axis_index("c")
    s = lax.axis_index("s")
    w = c * NS + s
    pltpu.sync_copy(z1_hbm.at[pl.ds(0, SLICE)], zbuf)
    for a_sh in (a0_sh, a1_sh, a2_sh, a3_sh):
        pltpu.sync_copy(zbuf, a_sh.at[pl.ds(s * SLICE, SLICE)])
    pltpu.sync_copy(ones_hbm, ones_v)
    plsc.subcore_barrier()
    base = w * EPW

    sidx = (sidx0, sidx1)
    didx = (didx0, didx1)
    g = ((g00, g10, g20), (g01, g11, g21))
    vt = (v0_hbm, v1_hbm, v2_hbm)
    semg = (semg0, semg1)
    sems = (sems0, sems1)
    ash = (a0_sh, a1_sh, a2_sh)
    gd = [None, None]
    sd = [None, None]

    for i in range(NCHUNK + 1):
        b = i % 2
        pb = 1 - b
        if i < NCHUNK:
            if sd[b] is not None:
                for dd in sd[b]:
                    dd.wait()
                sd[b] = None
            off = base + i * CH
            pltpu.sync_copy(src_hbm.at[pl.ds(off, CH)], sidx[b])
            pltpu.sync_copy(dst_hbm.at[pl.ds(off, CH)], didx[b])
            gd[b] = [pltpu.async_copy(vt[k].at[sidx[b]], g[b][k], semg[b])
                     for k in range(3)]
        if i > 0:
            for dd in gd[pb]:
                dd.wait()
            gd[pb] = None
            sd[pb] = [pltpu.async_copy(g[pb][k], ash[k].at[didx[pb]],
                                       sems[pb], add=True)
                      for k in range(3)]
            sd[pb].append(pltpu.async_copy(ones_v, a3_sh.at[didx[pb]],
                                           sems[pb], add=True))
    for b in range(2):
        if sd[b] is not None:
            for dd in sd[b]:
                dd.wait()

    plsc.subcore_barrier()
    for ch, a_sh in enumerate((a0_sh, a1_sh, a2_sh, a3_sh)):
        pltpu.sync_copy(a_sh.at[pl.ds(s * SLICE, SLICE)], zbuf)
        pltpu.sync_copy(zbuf,
                        agg_hbm.at[pl.ds((c * 4 + ch) * NPAD + s * SLICE,
                                         SLICE)])


def _scatter(src, dst, v0, v1, v2, z1, ones_c):
    mesh = plsc.VectorSubcoreMesh(core_axis_name="c", subcore_axis_name="s")
    f = pl.kernel(
        _k3_body,
        out_type=jax.ShapeDtypeStruct((NC * 4 * NPAD,), jnp.float32),
        mesh=mesh,
        scratch_types=(
            [pltpu.VMEM((CH,), jnp.int32)] * 4
            + [pltpu.VMEM((CH,), jnp.float32)] * 7
            + [pltpu.VMEM((SLICE,), jnp.float32)]
            + [pltpu.VMEM_SHARED((NPAD,), jnp.float32)] * 4
            + [pltpu.SemaphoreType.DMA] * 4
        ),
        compiler_params=_SC_PARAMS,
    )
    return f(src, dst, v0, v1, v2, z1, ones_c)


def _k2_body(ho0_ref, ho1_ref, nw_ref, sg_ref, emb_ref, v0_ref, v1_ref, v2_ref):
    od = jnp.maximum(ho0_ref[...] + ho1_ref[...], 1.0)
    sc = lax.rsqrt(od)
    sg = sg_ref[...]
    e00 = emb_ref[0, 0]
    e01 = emb_ref[0, 1]
    e10 = emb_ref[1, 0]
    e11 = emb_ref[1, 1]
    v0_ref[...] = nw_ref[...] * sc
    v1_ref[...] = (e00 + (e10 - e00) * sg) * sc
    v2_ref[...] = (e01 + (e11 - e01) * sg) * sc


ROWS = NPAD // DIM


def _table(ho0, ho1, nwp, sgp, emb):
    full = pl.BlockSpec((ROWS, DIM), lambda i: (0, 0))
    shp = jax.ShapeDtypeStruct((ROWS, DIM), jnp.float32)
    return pl.pallas_call(
        _k2_body,
        grid=(1,),
        in_specs=[full, full, full, full, pl.BlockSpec((2, 2), lambda i: (0, 0))],
        out_specs=[full, full, full],
        out_shape=[shp, shp, shp],
    )(ho0, ho1, nwp, sgp, emb)


PACK = 8


def _k4_body(q0_ref, f10_ref, f20_ref, c0_ref, q1_ref, f11_ref, f21_ref,
             c1_ref, w_ref, b_ref, o_ref):
    q = q0_ref[...] + q1_ref[...]
    f1 = f10_ref[...] + f11_ref[...]
    f2 = f20_ref[...] + f21_ref[...]
    cnt = c0_ref[...] + c1_ref[...]
    r = lax.rsqrt(jnp.maximum(cnt, 1.0))
    w = w_ref[...]
    t = ((q * r)[:, :, None] * w[0].reshape(1, 1, DIM)
         + (f1 * r)[:, :, None] * w[1].reshape(1, 1, DIM)
         + (f2 * r)[:, :, None] * w[2].reshape(1, 1, DIM)
         + b_ref[...].reshape(1, 1, DIM))
    o_ref[...] = t.reshape(RBLK, DIM)


def _project(parts, W0, b0r):
    pk = pl.BlockSpec((PACK, DIM), lambda i: (i, 0))
    return pl.pallas_call(
        _k4_body,
        grid=(NBLK,),
        in_specs=[pk] * 8 + [pl.BlockSpec((3, DIM), lambda i: (0, 0)),
                             pl.BlockSpec((1, DIM), lambda i: (0, 0))],
        out_specs=pl.BlockSpec((RBLK, DIM), lambda i: (i, 0)),
        out_shape=jax.ShapeDtypeStruct((N, DIM), jnp.float32),
    )(*parts, W0, b0r)


def kernel(significance, node_weight, edge_index, emb_table, W0, b0):
    src = edge_index[0].astype(jnp.int32)
    dst = edge_index[1].astype(jnp.int32)
    nwp = jnp.pad(node_weight.astype(jnp.float32), (0, NPAD - N)).reshape(ROWS, DIM)
    sgp = jnp.pad(significance.astype(jnp.float32), (0, NPAD - N)).reshape(ROWS, DIM)
    z1 = jnp.zeros((NPAD,), jnp.float32)
    ones_c = jnp.ones((CH,), jnp.float32)
    ones_c1 = jnp.ones((CH1,), jnp.float32)

    ho = _hist(src, z1, ones_c1)
    v0, v1, v2 = _table(ho[:NPAD].reshape(ROWS, DIM), ho[NPAD:].reshape(ROWS, DIM),
                        nwp, sgp, emb_table.astype(jnp.float32))
    agg = _scatter(src, dst, v0.reshape(NPAD), v1.reshape(NPAD),
                   v2.reshape(NPAD), z1, ones_c)
    a = agg.reshape(NC * 4, ROWS, DIM)
    out = _project([a[i] for i in range(8)],
                   W0.astype(jnp.float32),
                   b0.astype(jnp.float32).reshape(1, DIM))
    return out


# speedup vs baseline: 65.1731x; 1.7199x over previous
import jax
import jax.numpy as jnp
from jax import lax
from jax.experimental import pallas as pl
from jax.experimental.pallas import tpu as pltpu
from jax.experimental.pallas import tpu_sc as plsc

N = 100000
E = 3200000
DIM = 128
NC = 2
NS = 16
NW = NC * NS
NPAD = 100352
SLICE = NPAD // NS
EPW = E // NW
CH = 4000
CH1 = 50000
NCHUNK = EPW // CH
RBLK = 1024
NBLK = NPAD // RBLK

_SC_PARAMS = pltpu.CompilerParams(use_tc_tiling_on_sc=False)


def _k1_body(src_hbm, z1_hbm, ones_hbm, ho_hbm, ones_v, idx_v, zbuf, hist_sh):
    c = lax.axis_index("c")
    s = lax.axis_index("s")
    w = c * NS + s
    pltpu.sync_copy(z1_hbm.at[pl.ds(0, SLICE)], zbuf)
    pltpu.sync_copy(zbuf, hist_sh.at[pl.ds(s * SLICE, SLICE)])
    pltpu.sync_copy(ones_hbm, ones_v)
    plsc.subcore_barrier()
    base = w * EPW

    def chunk(i, carry):
        pltpu.sync_copy(src_hbm.at[pl.ds(base + i * CH1, CH1)], idx_v)
        pltpu.sync_copy(ones_v, hist_sh.at[idx_v], add=True)
        return carry

    lax.fori_loop(0, EPW // CH1, chunk, 0)
    plsc.subcore_barrier()
    pltpu.sync_copy(hist_sh.at[pl.ds(s * SLICE, SLICE)], zbuf)
    pltpu.sync_copy(zbuf, ho_hbm.at[pl.ds(c * NPAD + s * SLICE, SLICE)])


def _hist(src, z1, ones_c):
    mesh = plsc.VectorSubcoreMesh(core_axis_name="c", subcore_axis_name="s")
    f = pl.kernel(
        _k1_body,
        out_type=jax.ShapeDtypeStruct((NC * NPAD,), jnp.float32),
        mesh=mesh,
        scratch_types=[
            pltpu.VMEM((CH1,), jnp.float32),
            pltpu.VMEM((CH1,), jnp.int32),
            pltpu.VMEM((SLICE,), jnp.float32),
            pltpu.VMEM_SHARED((NPAD,), jnp.float32),
        ],
        compiler_params=_SC_PARAMS,
    )
    return f(src, z1, ones_c)


def _k3_body(src_hbm, dst_hbm, w_hbm, z1_hbm, zi1_hbm, ones_hbm,
             oq_hbm, osx_hbm, oc_hbm,
             sidx0, sidx1, didx0, didx1,
             wb0, wb1, qb0, qb1, xb0, xb1, ones_v, zbuf, zbufi,
             aq_sh, asx_sh, ac_sh,
             semg0, semg1, sems0, sems1):
    c = lax.axis_index("c")
    s = lax.axis_index("s")
    w = c * NS + s
    pltpu.sync_copy(z1_hbm.at[pl.ds(0, SLICE)], zbuf)
    pltpu.sync_copy(zi1_hbm.at[pl.ds(0, SLICE)], zbufi)
    pltpu.sync_copy(zbuf, aq_sh.at[pl.ds(s * SLICE, SLICE)])
    pltpu.sync_copy(zbuf, ac_sh.at[pl.ds(s * SLICE, SLICE)])
    pltpu.sync_copy(zbufi, asx_sh.at[pl.ds(s * SLICE, SLICE)])
    pltpu.sync_copy(ones_hbm, ones_v)
    plsc.subcore_barrier()
    base = w * EPW

    sidx = (sidx0, sidx1)
    didx = (didx0, didx1)
    wb = (wb0, wb1)
    qb = (qb0, qb1)
    xb = (xb0, xb1)
    semg = (semg0, semg1)
    sems = (sems0, sems1)
    gd = [None, None]
    sd = [None, None]
    m16 = jnp.int32(0xFFFF)
    mhi = jnp.int32(-65536)

    def unpack(b):
        def step(k, carry):
            ww = wb[b][pl.ds(k * 16, 16)]
            qb[b][pl.ds(k * 16, 16)] = lax.bitcast_convert_type(
                ww & mhi, jnp.float32)
            t = ww & m16
            sq = lax.shift_right_logical(t, 1)
            xq = sq * (t & 1)
            xb[b][pl.ds(k * 16, 16)] = lax.shift_left(sq, 16) | xq
            return carry
        lax.fori_loop(0, CH // 16, step, 0)

    for i in range(NCHUNK + 1):
        b = i % 2
        pb = 1 - b
        if i < NCHUNK:
            if sd[b] is not None:
                for dd in sd[b]:
                    dd.wait()
                sd[b] = None
            off = base + i * CH
            pltpu.sync_copy(src_hbm.at[pl.ds(off, CH)], sidx[b])
            pltpu.sync_copy(dst_hbm.at[pl.ds(off, CH)], didx[b])
            gd[b] = pltpu.async_copy(w_hbm.at[sidx[b]], wb[b], semg[b])
        if i > 0:
            gd[pb].wait()
            gd[pb] = None
            unpack(pb)
            sd[pb] = [
                pltpu.async_copy(qb[pb], aq_sh.at[didx[pb]], sems[pb],
                                 add=True),
                pltpu.async_copy(xb[pb], asx_sh.at[didx[pb]], sems[pb],
                                 add=True),
                pltpu.async_copy(ones_v, ac_sh.at[didx[pb]], sems[pb],
                                 add=True),
            ]
    for b in range(2):
        if sd[b] is not None:
            for dd in sd[b]:
                dd.wait()

    plsc.subcore_barrier()
    pltpu.sync_copy(aq_sh.at[pl.ds(s * SLICE, SLICE)], zbuf)
    pltpu.sync_copy(zbuf, oq_hbm.at[pl.ds(c * NPAD + s * SLICE, SLICE)])
    pltpu.sync_copy(ac_sh.at[pl.ds(s * SLICE, SLICE)], zbuf)
    pltpu.sync_copy(zbuf, oc_hbm.at[pl.ds(c * NPAD + s * SLICE, SLICE)])
    pltpu.sync_copy(asx_sh.at[pl.ds(s * SLICE, SLICE)], zbufi)
    pltpu.sync_copy(zbufi, osx_hbm.at[pl.ds(c * NPAD + s * SLICE, SLICE)])


def _scatter(src, dst, wtbl, z1, zi1, ones_c):
    mesh = plsc.VectorSubcoreMesh(core_axis_name="c", subcore_axis_name="s")
    f = pl.kernel(
        _k3_body,
        out_type=(jax.ShapeDtypeStruct((NC * NPAD,), jnp.float32),
                  jax.ShapeDtypeStruct((NC * NPAD,), jnp.int32),
                  jax.ShapeDtypeStruct((NC * NPAD,), jnp.float32)),
        mesh=mesh,
        scratch_types=(
            [pltpu.VMEM((CH,), jnp.int32)] * 4
            + [pltpu.VMEM((CH,), jnp.int32)] * 2
            + [pltpu.VMEM((CH,), jnp.float32)] * 2
            + [pltpu.VMEM((CH,), jnp.int32)] * 2
            + [pltpu.VMEM((CH,), jnp.float32)]
            + [pltpu.VMEM((SLICE,), jnp.float32),
               pltpu.VMEM((SLICE,), jnp.int32)]
            + [pltpu.VMEM_SHARED((NPAD,), jnp.float32),
               pltpu.VMEM_SHARED((NPAD,), jnp.int32),
               pltpu.VMEM_SHARED((NPAD,), jnp.float32)]
            + [pltpu.SemaphoreType.DMA] * 4
        ),
        compiler_params=_SC_PARAMS,
    )
    return f(src, dst, wtbl, z1, zi1, ones_c)


SQ = 512


def _k2_body(ho0_ref, ho1_ref, nw_ref, sg_ref, w_ref):
    od = jnp.maximum(ho0_ref[...] + ho1_ref[...], 1.0)
    sc = lax.rsqrt(od)
    q = nw_ref[...] * sc
    qb = lax.bitcast_convert_type(q, jnp.int32)
    qr = (qb + 0x7FFF + (lax.shift_right_logical(qb, 16) & 1)) & ~0xFFFF
    s_q = (sc * SQ + 0.5).astype(jnp.int32)
    sig = sg_ref[...].astype(jnp.int32)
    w_ref[...] = qr | lax.shift_left(s_q, 1) | sig


ROWS = NPAD // DIM


def _table(ho0, ho1, nwp, sgp):
    full = pl.BlockSpec((ROWS, DIM), lambda i: (0, 0))
    return pl.pallas_call(
        _k2_body,
        grid=(1,),
        in_specs=[full, full, full, full],
        out_specs=full,
        out_shape=jax.ShapeDtypeStruct((ROWS, DIM), jnp.int32),
    )(ho0, ho1, nwp, sgp)


PACK = 8


def _k4_body(q0_ref, q1_ref, sx0_ref, sx1_ref, c0_ref, c1_ref,
             emb_ref, w_ref, b_ref, o_ref):
    q = q0_ref[...] + q1_ref[...]
    cnt = c0_ref[...] + c1_ref[...]
    sxw = sx0_ref[...] + sx1_ref[...]
    m16 = jnp.int32(0xFFFF)
    inv = jnp.float32(1.0 / SQ)
    S = (lax.shift_right_logical(sxw, 16) & m16).astype(jnp.float32) * inv
    X = (sxw & m16).astype(jnp.float32) * inv
    e00 = emb_ref[0, 0]
    e01 = emb_ref[0, 1]
    e10 = emb_ref[1, 0]
    e11 = emb_ref[1, 1]
    f1 = e00 * S + (e10 - e00) * X
    f2 = e01 * S + (e11 - e01) * X
    r = lax.rsqrt(jnp.maximum(cnt, 1.0))
    w = w_ref[...]
    t = ((q * r)[:, :, None] * w[0].reshape(1, 1, DIM)
         + (f1 * r)[:, :, None] * w[1].reshape(1, 1, DIM)
         + (f2 * r)[:, :, None] * w[2].reshape(1, 1, DIM)
         + b_ref[...].reshape(1, 1, DIM))
    o_ref[...] = t.reshape(RBLK, DIM)


def _project(q0, q1, sx0, sx1, c0, c1, emb, W0, b0r):
    pk = pl.BlockSpec((PACK, DIM), lambda i: (i, 0))
    return pl.pallas_call(
        _k4_body,
        grid=(NBLK,),
        in_specs=[pk] * 6 + [pl.BlockSpec((2, 2), lambda i: (0, 0)),
                             pl.BlockSpec((3, DIM), lambda i: (0, 0)),
                             pl.BlockSpec((1, DIM), lambda i: (0, 0))],
        out_specs=pl.BlockSpec((RBLK, DIM), lambda i: (i, 0)),
        out_shape=jax.ShapeDtypeStruct((N, DIM), jnp.float32),
    )(q0, q1, sx0, sx1, c0, c1, emb, W0, b0r)


def kernel(significance, node_weight, edge_index, emb_table, W0, b0):
    src = edge_index[0].astype(jnp.int32)
    dst = edge_index[1].astype(jnp.int32)
    nwp = jnp.pad(node_weight.astype(jnp.float32), (0, NPAD - N)).reshape(ROWS, DIM)
    sgp = jnp.pad(significance.astype(jnp.float32), (0, NPAD - N)).reshape(ROWS, DIM)
    z1 = jnp.zeros((NPAD,), jnp.float32)
    zi1 = jnp.zeros((NPAD,), jnp.int32)
    ones_c = jnp.ones((CH,), jnp.float32)
    ones_c1 = jnp.ones((CH1,), jnp.float32)
    emb = emb_table.astype(jnp.float32)

    ho = _hist(src, z1, ones_c1)
    wtbl = _table(ho[:NPAD].reshape(ROWS, DIM), ho[NPAD:].reshape(ROWS, DIM),
                  nwp, sgp)
    oq, osx, oc = _scatter(src, dst, wtbl.reshape(NPAD), z1, zi1, ones_c)
    qp = oq.reshape(NC, ROWS, DIM)
    sxp = osx.reshape(NC, ROWS, DIM)
    cp = oc.reshape(NC, ROWS, DIM)
    out = _project(qp[0], qp[1], sxp[0], sxp[1], cp[0], cp[1], emb,
                   W0.astype(jnp.float32),
                   b0.astype(jnp.float32).reshape(1, DIM))
    return out
